# baseline (speedup 1.0000x reference)
import functools

import jax
import jax.numpy as jnp
from jax import lax
from jax.experimental import pallas as pl
from jax.experimental.pallas import tpu as pltpu
from jax.experimental.pallas import tpu_sc as plsc

NUM_CORES = 2
NUM_SUBCORES = 16
NUM_WORKERS = NUM_CORES * NUM_SUBCORES

BATCH = 4096
TEXT = 50
DIM = 128
NUM_SPLITS = 4
PART_B = BATCH // NUM_SPLITS


def _make_lookup(batch: int, text: int, dim: int, rows_per_chunk: int):
  assert batch % NUM_WORKERS == 0
  rows_per_w = batch // NUM_WORKERS
  assert rows_per_w % (2 * rows_per_chunk) == 0
  n_pairs = rows_per_w // (2 * rows_per_chunk)
  chunk = rows_per_chunk * text
  idx_per_w = rows_per_w * text
  assert chunk % 8 == 0

  mesh = plsc.VectorSubcoreMesh(core_axis_name="c", subcore_axis_name="s")

  @functools.partial(
      pl.kernel,
      mesh=mesh,
      out_type=jax.ShapeDtypeStruct((batch, text, dim), jnp.float32),
      scratch_types=[
          pltpu.VMEM((idx_per_w,), jnp.int32),
          pltpu.VMEM((chunk, dim), jnp.float32),
          pltpu.VMEM((chunk, dim), jnp.float32),
          pltpu.SemaphoreType.DMA,
          pltpu.SemaphoreType.DMA,
      ],
  )
  def lookup_kernel(table_hbm, idx_hbm, out_hbm, idx_v, buf0, buf1, sem0,
                    sem1):
    wid = lax.axis_index("s") * NUM_CORES + lax.axis_index("c")
    row_base = wid * rows_per_w
    pltpu.sync_copy(idx_hbm.at[pl.ds(row_base * text, idx_per_w)], idx_v)

    def gather_start(c, buf, sem):
      pltpu.async_copy(
          table_hbm.at[idx_v.at[pl.ds(c * chunk, chunk)]], buf, sem
      )

    def gather_wait(c, buf, sem):
      pltpu.make_async_copy(
          table_hbm.at[idx_v.at[pl.ds(c * chunk, chunk)]], buf, sem
      ).wait()

    def store(c, buf):
      row0 = row_base + c * rows_per_chunk
      for r in range(rows_per_chunk):
        pltpu.sync_copy(
            buf.at[pl.ds(r * text, text)], out_hbm.at[row0 + r]
        )

    gather_start(0, buf0, sem0)

    def body(p, carry):
      c0 = 2 * p
      gather_start(c0 + 1, buf1, sem1)
      gather_wait(c0, buf0, sem0)
      store(c0, buf0)

      @pl.when(p + 1 < n_pairs)
      def _():
        gather_start(c0 + 2, buf0, sem0)

      gather_wait(c0 + 1, buf1, sem1)
      store(c0 + 1, buf1)
      return carry

    lax.fori_loop(0, n_pairs, body, 0)

  return lookup_kernel


_lookup_part = _make_lookup(PART_B, TEXT, DIM, 8)


def kernel(input, table):
  idx = input.astype(jnp.int32)
  parts = [
      _lookup_part(table, idx[k * PART_B:(k + 1) * PART_B].reshape(-1))
      for k in range(NUM_SPLITS)
  ]
  acc = jnp.pad(parts[0], ((0, BATCH - PART_B), (0, 0), (0, 0)))
  for k in range(1, NUM_SPLITS):
    acc = lax.dynamic_update_slice(
        acc, parts[k], (k * PART_B, 0, 0)
    )
  return acc


# speedup vs baseline: 1.7509x; 1.7509x over previous
import functools

import jax
import jax.numpy as jnp
from jax import lax
from jax.experimental import pallas as pl
from jax.experimental.pallas import tpu as pltpu
from jax.experimental.pallas import tpu_sc as plsc

NUM_CORES = 2
NUM_SUBCORES = 16
NUM_WORKERS = NUM_CORES * NUM_SUBCORES
NBUF = 4


def _make_lookup(batch: int, text: int, dim: int, rows_per_chunk: int):
  assert batch % NUM_WORKERS == 0
  rows_per_w = batch // NUM_WORKERS
  assert rows_per_w % (NBUF * rows_per_chunk) == 0
  chunk = rows_per_chunk * text
  n_chunks = rows_per_w // rows_per_chunk
  n_groups = n_chunks // NBUF
  idx_per_w = rows_per_w * text
  assert chunk % 8 == 0

  mesh = plsc.VectorSubcoreMesh(core_axis_name="c", subcore_axis_name="s")

  @functools.partial(
      pl.kernel,
      mesh=mesh,
      out_type=jax.ShapeDtypeStruct((batch, text, dim), jnp.float32),
      scratch_types=[
          pltpu.VMEM((idx_per_w,), jnp.int32),
          [pltpu.VMEM((chunk, dim), jnp.float32) for _ in range(NBUF)],
          [pltpu.SemaphoreType.DMA for _ in range(NBUF)],
      ],
  )
  def lookup_kernel(table_hbm, idx_hbm, out_hbm, idx_v, bufs, sems):
    wid = lax.axis_index("s") * NUM_CORES + lax.axis_index("c")
    row_base = wid * rows_per_w
    pltpu.sync_copy(idx_hbm.at[pl.ds(row_base * text, idx_per_w)], idx_v)

    def gather_start(c, j):
      pltpu.async_copy(
          table_hbm.at[idx_v.at[pl.ds(c * chunk, chunk)]], bufs[j], sems[j]
      )

    def gather_wait(c, j):
      pltpu.make_async_copy(
          table_hbm.at[idx_v.at[pl.ds(c * chunk, chunk)]], bufs[j], sems[j]
      ).wait()

    def store(c, j):
      row0 = row_base + c * rows_per_chunk
      for r in range(rows_per_chunk):
        pltpu.sync_copy(
            bufs[j].at[pl.ds(r * text, text)], out_hbm.at[row0 + r]
        )

    for j in range(NBUF - 1):
      gather_start(j, j)

    def body(g, carry):
      c0 = g * NBUF
      for j in range(NBUF):
        c = c0 + j
        nxt = c + NBUF - 1
        jn = (j + NBUF - 1) % NBUF
        @pl.when(nxt < n_chunks)
        def _(nxt=nxt, jn=jn):
          gather_start(nxt, jn)
        gather_wait(c, j)
        store(c, j)
      return carry

    lax.fori_loop(0, n_groups, body, 0)

  return lookup_kernel


_lookup = _make_lookup(4096, 50, 128, 4)


def kernel(input, table):
  idx = input.reshape(-1).astype(jnp.int32)
  return _lookup(table, idx)
